# baseline (speedup 1.0000x reference)
import jax
import jax.numpy as jnp
from jax.experimental import pallas as pl
from jax.experimental.pallas import tpu as pltpu

_N = 100000
_F = 128
_K = 1000
_ROWS_PER_BLK = 2000


def _scores_body(emb_ref, scorer_ref, out_ref):
    s = scorer_ref[...]
    nrm = jnp.sqrt(jnp.sum(s * s))
    out_ref[...] = jnp.dot(
        emb_ref[...], s, preferred_element_type=jnp.float32
    ) / nrm


def _gather_body(idx_ref, emb_ref, val_ref, out_ref):
    del idx_ref
    out_ref[...] = emb_ref[...] * jnp.tanh(val_ref[0, 0, 0])


def kernel(node_embs, scorer):
    scores = pl.pallas_call(
        _scores_body,
        grid=(_N // _ROWS_PER_BLK,),
        in_specs=[
            pl.BlockSpec((_ROWS_PER_BLK, _F), lambda i: (i, 0)),
            pl.BlockSpec((_F, 1), lambda i: (0, 0)),
        ],
        out_specs=pl.BlockSpec((_ROWS_PER_BLK, 1), lambda i: (i, 0)),
        out_shape=jax.ShapeDtypeStruct((_N, 1), jnp.float32),
    )(node_embs, scorer)

    vals, topk_idx = jax.lax.top_k(scores.reshape(-1), _K)

    sel = pl.pallas_call(
        _gather_body,
        grid_spec=pltpu.PrefetchScalarGridSpec(
            num_scalar_prefetch=1,
            grid=(_K,),
            in_specs=[
                pl.BlockSpec((1, 1, _F), lambda k, idx_ref: (idx_ref[k], 0, 0)),
                pl.BlockSpec((1, 1, 1), lambda k, idx_ref: (k, 0, 0)),
            ],
            out_specs=pl.BlockSpec((1, 1, _F), lambda k, idx_ref: (k, 0, 0)),
        ),
        out_shape=jax.ShapeDtypeStruct((_K, 1, _F), jnp.float32),
    )(topk_idx, node_embs.reshape(_N, 1, _F), vals.reshape(_K, 1, 1))

    return sel.reshape(_K, _F).T


# speedup vs baseline: 2.6760x; 2.6760x over previous
import functools

import jax
import jax.numpy as jnp
from jax.experimental import pallas as pl
from jax.experimental.pallas import tpu as pltpu
from jax.experimental.pallas import tpu_sc as plsc

_N = 100000
_F = 128
_K = 1000
_ROWS_PER_BLK = 2000

_NC = 2
_NS = 16
_NW = _NC * _NS
_B = 1024
_BPW = _B // _NW


def _scores_body(emb_ref, scorer_ref, out_ref):
    s = scorer_ref[...]
    nrm = jnp.sqrt(jnp.sum(s * s))
    out_ref[...] = jnp.dot(
        emb_ref[...], s, preferred_element_type=jnp.float32
    ) / nrm


@functools.partial(
    pl.kernel,
    mesh=plsc.VectorSubcoreMesh(core_axis_name="c", subcore_axis_name="s"),
    out_type=jax.ShapeDtypeStruct((_B, _F), jnp.float32),
    scratch_types=[
        pltpu.VMEM((_BPW,), jnp.int32),
        pltpu.VMEM((_BPW, _F), jnp.float32),
        pltpu.SemaphoreType.DMA,
    ],
)
def _sc_gather(table_hbm, idx_hbm, out_hbm, idx_v, rows_v, sem):
    wid = jax.lax.axis_index("s") * _NC + jax.lax.axis_index("c")
    base = wid * _BPW
    pltpu.sync_copy(idx_hbm.at[pl.ds(base, _BPW)], idx_v)
    pltpu.async_copy(table_hbm.at[idx_v], rows_v, sem).wait()
    pltpu.sync_copy(rows_v, out_hbm.at[pl.ds(base, _BPW)])


def _finish_body(sel_ref, vals_ref, out_ref):
    scaled = sel_ref[...] * jnp.tanh(vals_ref[...])
    out_ref[...] = scaled.T


def kernel(node_embs, scorer):
    scores = pl.pallas_call(
        _scores_body,
        grid=(_N // _ROWS_PER_BLK,),
        in_specs=[
            pl.BlockSpec((_ROWS_PER_BLK, _F), lambda i: (i, 0)),
            pl.BlockSpec((_F, 1), lambda i: (0, 0)),
        ],
        out_specs=pl.BlockSpec((_ROWS_PER_BLK, 1), lambda i: (i, 0)),
        out_shape=jax.ShapeDtypeStruct((_N, 1), jnp.float32),
    )(node_embs, scorer)

    vals, topk_idx = jax.lax.top_k(scores.reshape(-1), _K)
    idx_pad = jnp.concatenate(
        [topk_idx, jnp.zeros((_B - _K,), dtype=topk_idx.dtype)]
    )
    vals_pad = jnp.concatenate([vals, jnp.zeros((_B - _K,), dtype=vals.dtype)])

    sel = _sc_gather(node_embs, idx_pad)

    out = pl.pallas_call(
        _finish_body,
        in_specs=[
            pl.BlockSpec((_B, _F), lambda: (0, 0)),
            pl.BlockSpec((_B, 1), lambda: (0, 0)),
        ],
        out_specs=pl.BlockSpec((_F, _B), lambda: (0, 0)),
        out_shape=jax.ShapeDtypeStruct((_F, _B), jnp.float32),
    )(sel, vals_pad.reshape(_B, 1))

    return out[:, :_K]


# speedup vs baseline: 2.9428x; 1.0997x over previous
import functools

import jax
import jax.numpy as jnp
from jax.experimental import pallas as pl
from jax.experimental.pallas import tpu as pltpu
from jax.experimental.pallas import tpu_sc as plsc

_N = 100000
_F = 128
_K = 1000
_ROWS_PER_BLK = 10000

_NC = 2
_NS = 16
_NW = _NC * _NS
_B = 1024
_BPW = _B // _NW


def _scores_body(emb_ref, scorer_ref, out_ref):
    s = scorer_ref[...]
    nrm = jnp.sqrt(jnp.sum(s * s))
    out_ref[...] = jnp.dot(
        emb_ref[...], s, preferred_element_type=jnp.float32
    ) / nrm


@functools.partial(
    pl.kernel,
    mesh=plsc.VectorSubcoreMesh(core_axis_name="c", subcore_axis_name="s"),
    out_type=jax.ShapeDtypeStruct((_B, _F), jnp.float32),
    scratch_types=[
        pltpu.VMEM((_BPW,), jnp.int32),
        pltpu.VMEM((_BPW, _F), jnp.float32),
        pltpu.SemaphoreType.DMA,
    ],
)
def _sc_gather(table_hbm, idx_hbm, out_hbm, idx_v, rows_v, sem):
    wid = jax.lax.axis_index("s") * _NC + jax.lax.axis_index("c")
    base = wid * _BPW
    pltpu.sync_copy(idx_hbm.at[pl.ds(base, _BPW)], idx_v)
    pltpu.async_copy(table_hbm.at[idx_v], rows_v, sem).wait()
    pltpu.sync_copy(rows_v, out_hbm.at[pl.ds(base, _BPW)])


def _finish_body(sel_ref, vals_ref, out_ref):
    scaled = sel_ref[...] * jnp.tanh(vals_ref[...])
    out_ref[...] = scaled.T


def kernel(node_embs, scorer):
    scores = pl.pallas_call(
        _scores_body,
        grid=(_N // _ROWS_PER_BLK,),
        in_specs=[
            pl.BlockSpec((_ROWS_PER_BLK, _F), lambda i: (i, 0)),
            pl.BlockSpec((_F, 1), lambda i: (0, 0)),
        ],
        out_specs=pl.BlockSpec((_ROWS_PER_BLK, 1), lambda i: (i, 0)),
        out_shape=jax.ShapeDtypeStruct((_N, 1), jnp.float32),
    )(node_embs, scorer)

    vals, topk_idx = jax.lax.top_k(scores.reshape(-1), _K)
    idx_pad = jnp.concatenate(
        [topk_idx, jnp.zeros((_B - _K,), dtype=topk_idx.dtype)]
    )
    vals_pad = jnp.concatenate([vals, jnp.zeros((_B - _K,), dtype=vals.dtype)])

    sel = _sc_gather(node_embs, idx_pad)

    out = pl.pallas_call(
        _finish_body,
        in_specs=[
            pl.BlockSpec((_B, _F), lambda: (0, 0)),
            pl.BlockSpec((_B, 1), lambda: (0, 0)),
        ],
        out_specs=pl.BlockSpec((_F, _B), lambda: (0, 0)),
        out_shape=jax.ShapeDtypeStruct((_F, _B), jnp.float32),
    )(sel, vals_pad.reshape(_B, 1))

    return out[:, :_K]


# speedup vs baseline: 2.9808x; 1.0129x over previous
import functools

import jax
import jax.numpy as jnp
from jax.experimental import pallas as pl
from jax.experimental.pallas import tpu as pltpu
from jax.experimental.pallas import tpu_sc as plsc

_N = 100000
_F = 128
_K = 1000
_ROWS_PER_BLK = 20000

_NC = 2
_NS = 16
_NW = _NC * _NS
_B = 1024
_BPW = _B // _NW


def _scores_body(emb_ref, scorer_ref, out_ref):
    s = scorer_ref[...]
    nrm = jnp.sqrt(jnp.sum(s * s))
    out_ref[...] = jnp.dot(
        emb_ref[...], s, preferred_element_type=jnp.float32
    ) / nrm


@functools.partial(
    pl.kernel,
    mesh=plsc.VectorSubcoreMesh(core_axis_name="c", subcore_axis_name="s"),
    out_type=jax.ShapeDtypeStruct((_B, _F), jnp.float32),
    scratch_types=[
        pltpu.VMEM((_BPW,), jnp.int32),
        pltpu.VMEM((_BPW, _F), jnp.float32),
        pltpu.SemaphoreType.DMA,
    ],
)
def _sc_gather(table_hbm, idx_hbm, out_hbm, idx_v, rows_v, sem):
    wid = jax.lax.axis_index("s") * _NC + jax.lax.axis_index("c")
    base = wid * _BPW
    pltpu.sync_copy(idx_hbm.at[pl.ds(base, _BPW)], idx_v)
    pltpu.async_copy(table_hbm.at[idx_v], rows_v, sem).wait()
    pltpu.sync_copy(rows_v, out_hbm.at[pl.ds(base, _BPW)])


def _finish_body(sel_ref, vals_ref, out_ref):
    scaled = sel_ref[...] * jnp.tanh(vals_ref[...])
    out_ref[...] = scaled.T


def kernel(node_embs, scorer):
    scores = pl.pallas_call(
        _scores_body,
        grid=(_N // _ROWS_PER_BLK,),
        in_specs=[
            pl.BlockSpec((_ROWS_PER_BLK, _F), lambda i: (i, 0)),
            pl.BlockSpec((_F, 1), lambda i: (0, 0)),
        ],
        out_specs=pl.BlockSpec((_ROWS_PER_BLK, 1), lambda i: (i, 0)),
        out_shape=jax.ShapeDtypeStruct((_N, 1), jnp.float32),
    )(node_embs, scorer)

    vals, topk_idx = jax.lax.top_k(scores.reshape(-1), _K)
    idx_pad = jnp.concatenate(
        [topk_idx, jnp.zeros((_B - _K,), dtype=topk_idx.dtype)]
    )
    vals_pad = jnp.concatenate([vals, jnp.zeros((_B - _K,), dtype=vals.dtype)])

    sel = _sc_gather(node_embs, idx_pad)

    out = pl.pallas_call(
        _finish_body,
        in_specs=[
            pl.BlockSpec((_B, _F), lambda: (0, 0)),
            pl.BlockSpec((_B, 1), lambda: (0, 0)),
        ],
        out_specs=pl.BlockSpec((_F, _B), lambda: (0, 0)),
        out_shape=jax.ShapeDtypeStruct((_F, _B), jnp.float32),
    )(sel, vals_pad.reshape(_B, 1))

    return out[:, :_K]
